# baseline (speedup 1.0000x reference)
import functools

import jax
import jax.numpy as jnp
from jax import lax
from jax.experimental import pallas as pl
from jax.experimental.pallas import tpu as pltpu
from jax.experimental.pallas import tpu_sc as plsc

EMB_D = 128
BATCH = 16384
NUM_CORES = 2
NUM_SUBCORES = 16
NUM_WORKERS = NUM_CORES * NUM_SUBCORES
B_PER_W = BATCH // NUM_WORKERS
LANES = 16
CHUNKS = EMB_D // LANES
GROUPS = B_PER_W // LANES
NB = 4
GROUPS_PER_NB = GROUPS // NB
WORDS_PER_NB = B_PER_W * EMB_D // NB


def _build():
    mesh = plsc.VectorSubcoreMesh(core_axis_name="c", subcore_axis_name="s")

    @functools.partial(
        pl.kernel,
        mesh=mesh,
        out_type=jax.ShapeDtypeStruct((BATCH * EMB_D,), jnp.float32),
        scratch_types=[
            pltpu.VMEM((B_PER_W,), jnp.int32),
            pltpu.VMEM((3 * EMB_D,), jnp.float32),
            pltpu.VMEM((B_PER_W * EMB_D,), jnp.float32),
            pltpu.SemaphoreType.DMA,
            pltpu.SemaphoreType.DMA,
        ],
    )
    def lookup_kernel(labels_hbm, table_hbm, out_hbm, idx_v, tab_v, rows_v,
                      sem_in, sem_out):
        wid = lax.axis_index("s") * NUM_CORES + lax.axis_index("c")
        base = wid * B_PER_W
        cp_lab = pltpu.async_copy(
            labels_hbm.at[pl.ds(base, B_PER_W)], idx_v, sem_in)
        cp_tab = pltpu.async_copy(table_hbm, tab_v, sem_in)
        cp_lab.wait()
        cp_tab.wait()
        rows = [
            [tab_v[pl.ds(v * EMB_D + LANES * c, LANES)] for c in range(CHUNKS)]
            for v in range(3)
        ]
        d1 = [rows[1][c] - rows[0][c] for c in range(CHUNKS)]
        d2 = [rows[2][c] - rows[0][c] for c in range(CHUNKS)]

        def body(g):
            lbl16 = idx_v[pl.ds(g * LANES, LANES)]
            for j in range(LANES):
                lbl = lbl16[j]
                w1 = jnp.broadcast_to((lbl == 1).astype(jnp.float32), (LANES,))
                w2 = jnp.broadcast_to((lbl == 2).astype(jnp.float32), (LANES,))
                e = g * LANES + j
                for c in range(CHUNKS):
                    val = rows[0][c] + w1 * d1[c] + w2 * d2[c]
                    rows_v[pl.ds(e * EMB_D + LANES * c, LANES)] = val

        out_cps = []
        for nb in range(NB):
            plsc.parallel_loop(
                nb * GROUPS_PER_NB, (nb + 1) * GROUPS_PER_NB)(body)
            out_cps.append(pltpu.async_copy(
                rows_v.at[pl.ds(nb * WORDS_PER_NB, WORDS_PER_NB)],
                out_hbm.at[pl.ds(base * EMB_D + nb * WORDS_PER_NB, WORDS_PER_NB)],
                sem_out))
        for cp in out_cps:
            cp.wait()

    return lookup_kernel


_lookup = _build()


def kernel(labels, table):
    out = _lookup(labels, table.reshape(-1))
    return out.reshape(BATCH, 1, EMB_D)


# speedup vs baseline: 1.5762x; 1.5762x over previous
import functools

import jax
import jax.numpy as jnp
from jax import lax
from jax.experimental import pallas as pl
from jax.experimental.pallas import tpu as pltpu
from jax.experimental.pallas import tpu_sc as plsc

EMB_D = 128
BATCH = 16384
NUM_CORES = 2
NUM_SUBCORES = 16
NUM_WORKERS = NUM_CORES * NUM_SUBCORES
B_PER_W = BATCH // NUM_WORKERS
LANES = 16
CHUNKS = EMB_D // LANES
GROUPS = B_PER_W // LANES
NB = 4
GROUPS_PER_NB = GROUPS // NB
WORDS_PER_NB = B_PER_W * EMB_D // NB


def _build():
    mesh = plsc.VectorSubcoreMesh(core_axis_name="c", subcore_axis_name="s")

    @functools.partial(
        pl.kernel,
        mesh=mesh,
        out_type=jax.ShapeDtypeStruct((BATCH * EMB_D,), jnp.float32),
        scratch_types=[
            pltpu.VMEM((B_PER_W,), jnp.int32),
            pltpu.VMEM((3 * EMB_D,), jnp.float32),
            pltpu.VMEM((B_PER_W * EMB_D,), jnp.float32),
            pltpu.SemaphoreType.DMA,
            pltpu.SemaphoreType.DMA,
        ],
    )
    def lookup_kernel(labels_hbm, table_hbm, out_hbm, idx_v, tab_v, rows_v,
                      sem_in, sem_out):
        wid = lax.axis_index("s") * NUM_CORES + lax.axis_index("c")
        base = wid * B_PER_W
        cp_lab = pltpu.async_copy(
            labels_hbm.at[pl.ds(base, B_PER_W)], idx_v, sem_in)
        cp_tab = pltpu.async_copy(table_hbm, tab_v, sem_in)
        cp_lab.wait()
        cp_tab.wait()
        rows = [
            [tab_v[pl.ds(v * EMB_D + LANES * c, LANES)] for c in range(CHUNKS)]
            for v in range(3)
        ]
        d1 = [rows[1][c] - rows[0][c] for c in range(CHUNKS)]
        d2 = [rows[2][c] - rows[0][c] for c in range(CHUNKS)]

        def body(g, carry):
            lbl16 = idx_v[pl.ds(g * LANES, LANES)]
            lblf = lbl16.astype(jnp.float32)
            w1v = 1.0 - jnp.abs(lblf - 1.0)
            w2v = jnp.maximum(lblf - 1.0, 0.0)
            for j in range(LANES):
                w1 = jnp.broadcast_to(w1v[j], (LANES,))
                w2 = jnp.broadcast_to(w2v[j], (LANES,))
                e = g * LANES + j
                for c in range(CHUNKS):
                    val = rows[0][c] + w1 * d1[c] + w2 * d2[c]
                    rows_v[pl.ds(e * EMB_D + LANES * c, LANES)] = val
            return carry

        out_cps = []
        for nb in range(NB):
            lax.fori_loop(nb * GROUPS_PER_NB, (nb + 1) * GROUPS_PER_NB,
                          body, 0, unroll=2)
            out_cps.append(pltpu.async_copy(
                rows_v.at[pl.ds(nb * WORDS_PER_NB, WORDS_PER_NB)],
                out_hbm.at[pl.ds(base * EMB_D + nb * WORDS_PER_NB, WORDS_PER_NB)],
                sem_out))
        for cp in out_cps:
            cp.wait()

    return lookup_kernel


_lookup = _build()


def kernel(labels, table):
    out = _lookup(labels, table.reshape(-1))
    return out.reshape(BATCH, 1, EMB_D)


# speedup vs baseline: 1.6025x; 1.0167x over previous
import functools

import jax
import jax.numpy as jnp
from jax import lax
from jax.experimental import pallas as pl
from jax.experimental.pallas import tpu as pltpu
from jax.experimental.pallas import tpu_sc as plsc

EMB_D = 128
BATCH = 16384
NUM_CORES = 2
NUM_SUBCORES = 16
NUM_WORKERS = NUM_CORES * NUM_SUBCORES
B_PER_W = BATCH // NUM_WORKERS
LANES = 16
CHUNKS = EMB_D // LANES
GROUPS = B_PER_W // LANES
NB = 4
GROUPS_PER_NB = GROUPS // NB
WORDS_PER_NB = B_PER_W * EMB_D // NB


def _build():
    mesh = plsc.VectorSubcoreMesh(core_axis_name="c", subcore_axis_name="s")

    @functools.partial(
        pl.kernel,
        mesh=mesh,
        out_type=jax.ShapeDtypeStruct((BATCH * EMB_D,), jnp.float32),
        scratch_types=[
            pltpu.VMEM((B_PER_W,), jnp.int32),
            pltpu.VMEM((3 * EMB_D,), jnp.float32),
            pltpu.VMEM((B_PER_W * EMB_D,), jnp.float32),
            pltpu.SemaphoreType.DMA,
            pltpu.SemaphoreType.DMA,
        ],
    )
    def lookup_kernel(labels_hbm, table_hbm, out_hbm, idx_v, tab_v, rows_v,
                      sem_in, sem_out):
        wid = lax.axis_index("s") * NUM_CORES + lax.axis_index("c")
        base = wid * B_PER_W
        cp_lab = pltpu.async_copy(
            labels_hbm.at[pl.ds(base, B_PER_W)], idx_v, sem_in)
        cp_tab = pltpu.async_copy(table_hbm, tab_v, sem_in)
        cp_lab.wait()
        cp_tab.wait()
        rows = [
            [tab_v[pl.ds(v * EMB_D + LANES * c, LANES)] for c in range(CHUNKS)]
            for v in range(3)
        ]
        d1 = [rows[1][c] - rows[0][c] for c in range(CHUNKS)]
        d2 = [rows[2][c] - rows[0][c] for c in range(CHUNKS)]

        def body(g, carry):
            lbl16 = idx_v[pl.ds(g * LANES, LANES)]
            lblf = lbl16.astype(jnp.float32)
            w1v = 1.0 - jnp.abs(lblf - 1.0)
            w2v = jnp.maximum(lblf - 1.0, 0.0)
            for j in range(LANES):
                w1 = jnp.broadcast_to(w1v[j], (LANES,))
                w2 = jnp.broadcast_to(w2v[j], (LANES,))
                e = g * LANES + j
                for c in range(CHUNKS):
                    val = rows[0][c] + w1 * d1[c] + w2 * d2[c]
                    rows_v[pl.ds(e * EMB_D + LANES * c, LANES)] = val
            return carry

        out_cps = []
        for nb in range(NB):
            lax.fori_loop(nb * GROUPS_PER_NB, (nb + 1) * GROUPS_PER_NB,
                          body, 0)
            out_cps.append(pltpu.async_copy(
                rows_v.at[pl.ds(nb * WORDS_PER_NB, WORDS_PER_NB)],
                out_hbm.at[pl.ds(base * EMB_D + nb * WORDS_PER_NB, WORDS_PER_NB)],
                sem_out))
        for cp in out_cps:
            cp.wait()

    return lookup_kernel


_lookup = _build()


def kernel(labels, table):
    out = _lookup(labels, table.reshape(-1))
    return out.reshape(BATCH, 1, EMB_D)


# speedup vs baseline: 1.6584x; 1.0349x over previous
import functools

import jax
import jax.numpy as jnp
from jax import lax
from jax.experimental import pallas as pl
from jax.experimental.pallas import tpu as pltpu
from jax.experimental.pallas import tpu_sc as plsc

EMB_D = 128
BATCH = 16384
NUM_CORES = 2
NUM_SUBCORES = 16
NUM_WORKERS = NUM_CORES * NUM_SUBCORES
B_PER_W = BATCH // NUM_WORKERS
LANES = 16
CHUNKS = EMB_D // LANES
GROUPS = B_PER_W // LANES
NB = 8
GROUPS_PER_NB = GROUPS // NB
WORDS_PER_NB = B_PER_W * EMB_D // NB


def _build():
    mesh = plsc.VectorSubcoreMesh(core_axis_name="c", subcore_axis_name="s")

    @functools.partial(
        pl.kernel,
        mesh=mesh,
        out_type=jax.ShapeDtypeStruct((BATCH * EMB_D,), jnp.float32),
        scratch_types=[
            pltpu.VMEM((B_PER_W,), jnp.int32),
            pltpu.VMEM((3 * EMB_D,), jnp.float32),
            pltpu.VMEM((B_PER_W * EMB_D,), jnp.float32),
            pltpu.SemaphoreType.DMA,
            pltpu.SemaphoreType.DMA,
        ],
    )
    def lookup_kernel(labels_hbm, table_hbm, out_hbm, idx_v, tab_v, rows_v,
                      sem_in, sem_out):
        wid = lax.axis_index("s") * NUM_CORES + lax.axis_index("c")
        base = wid * B_PER_W
        cp_lab = pltpu.async_copy(
            labels_hbm.at[pl.ds(base, B_PER_W)], idx_v, sem_in)
        cp_tab = pltpu.async_copy(table_hbm, tab_v, sem_in)
        cp_lab.wait()
        cp_tab.wait()
        rows = [
            [tab_v[pl.ds(v * EMB_D + LANES * c, LANES)] for c in range(CHUNKS)]
            for v in range(3)
        ]
        d1 = [rows[1][c] - rows[0][c] for c in range(CHUNKS)]
        d2 = [rows[2][c] - rows[0][c] for c in range(CHUNKS)]

        def body(g, carry):
            lbl16 = idx_v[pl.ds(g * LANES, LANES)]
            for j in range(LANES):
                lbl = lbl16[j]
                w1 = jnp.broadcast_to((lbl == 1).astype(jnp.float32), (LANES,))
                w2 = jnp.broadcast_to((lbl == 2).astype(jnp.float32), (LANES,))
                e = g * LANES + j
                for c in range(CHUNKS):
                    val = rows[0][c] + w1 * d1[c] + w2 * d2[c]
                    rows_v[pl.ds(e * EMB_D + LANES * c, LANES)] = val
            return carry

        out_cps = []
        for nb in range(NB):
            lax.fori_loop(nb * GROUPS_PER_NB, (nb + 1) * GROUPS_PER_NB,
                          body, 0)
            out_cps.append(pltpu.async_copy(
                rows_v.at[pl.ds(nb * WORDS_PER_NB, WORDS_PER_NB)],
                out_hbm.at[pl.ds(base * EMB_D + nb * WORDS_PER_NB, WORDS_PER_NB)],
                sem_out))
        for cp in out_cps:
            cp.wait()

    return lookup_kernel


_lookup = _build()


def kernel(labels, table):
    out = _lookup(labels, table.reshape(-1))
    return out.reshape(BATCH, 1, EMB_D)


# speedup vs baseline: 1.7814x; 1.0741x over previous
import functools

import jax
import jax.numpy as jnp
from jax import lax
from jax.experimental import pallas as pl
from jax.experimental.pallas import tpu as pltpu
from jax.experimental.pallas import tpu_sc as plsc

EMB_D = 128
BATCH = 16384
NUM_CORES = 2
NUM_SUBCORES = 16
NUM_WORKERS = NUM_CORES * NUM_SUBCORES
B_PER_W = BATCH // NUM_WORKERS
LANES = 16
CHUNKS = EMB_D // LANES
GROUPS = B_PER_W // LANES
NB = 2
GROUPS_PER_NB = GROUPS // NB
WORDS_PER_NB = B_PER_W * EMB_D // NB


def _build():
    mesh = plsc.VectorSubcoreMesh(core_axis_name="c", subcore_axis_name="s")

    @functools.partial(
        pl.kernel,
        mesh=mesh,
        out_type=jax.ShapeDtypeStruct((BATCH * EMB_D,), jnp.float32),
        scratch_types=[
            pltpu.VMEM((B_PER_W,), jnp.int32),
            pltpu.VMEM((3 * EMB_D,), jnp.float32),
            pltpu.VMEM((B_PER_W * EMB_D,), jnp.float32),
            pltpu.SemaphoreType.DMA,
            pltpu.SemaphoreType.DMA,
        ],
    )
    def lookup_kernel(labels_hbm, table_hbm, out_hbm, idx_v, tab_v, rows_v,
                      sem_in, sem_out):
        wid = lax.axis_index("s") * NUM_CORES + lax.axis_index("c")
        base = wid * B_PER_W
        cp_lab = pltpu.async_copy(
            labels_hbm.at[pl.ds(base, B_PER_W)], idx_v, sem_in)
        cp_tab = pltpu.async_copy(table_hbm, tab_v, sem_in)
        cp_lab.wait()
        cp_tab.wait()
        rows = [
            [tab_v[pl.ds(v * EMB_D + LANES * c, LANES)] for c in range(CHUNKS)]
            for v in range(3)
        ]
        d1 = [rows[1][c] - rows[0][c] for c in range(CHUNKS)]
        d2 = [rows[2][c] - rows[0][c] for c in range(CHUNKS)]

        def body(g, carry):
            lbl16 = idx_v[pl.ds(g * LANES, LANES)]
            for j in range(LANES):
                lbl = lbl16[j]
                w1 = jnp.broadcast_to((lbl == 1).astype(jnp.float32), (LANES,))
                w2 = jnp.broadcast_to((lbl == 2).astype(jnp.float32), (LANES,))
                e = g * LANES + j
                for c in range(CHUNKS):
                    val = rows[0][c] + w1 * d1[c] + w2 * d2[c]
                    rows_v[pl.ds(e * EMB_D + LANES * c, LANES)] = val
            return carry

        out_cps = []
        for nb in range(NB):
            lax.fori_loop(nb * GROUPS_PER_NB, (nb + 1) * GROUPS_PER_NB,
                          body, 0)
            out_cps.append(pltpu.async_copy(
                rows_v.at[pl.ds(nb * WORDS_PER_NB, WORDS_PER_NB)],
                out_hbm.at[pl.ds(base * EMB_D + nb * WORDS_PER_NB, WORDS_PER_NB)],
                sem_out))
        for cp in out_cps:
            cp.wait()

    return lookup_kernel


_lookup = _build()


def kernel(labels, table):
    out = _lookup(labels, table.reshape(-1))
    return out.reshape(BATCH, 1, EMB_D)
